# Initial kernel scaffold; baseline (speedup 1.0000x reference)
import jax
import jax.numpy as jnp
from jax.experimental import pallas as pl


def kernel(x, codes):
    raise NotImplementedError("write your pallas kernel here")


# speedup vs baseline: 3953.5038x; 3953.5038x over previous
import functools

import jax
import jax.numpy as jnp
from jax import lax
from jax.experimental import pallas as pl
from jax.experimental.pallas import tpu as pltpu
from jax.experimental.pallas import tpu_sc as plsc

NUM_CODES = 128
LANES = 16


@functools.lru_cache(maxsize=None)
def _build_sc_kernel(n_total: int, chunk: int):
    info = plsc.get_sparse_core_info()
    nc, ns = info.num_cores, info.num_subcores
    nw = nc * ns
    per_w = n_total // nw
    assert n_total % nw == 0 and per_w % chunk == 0
    n_chunks = per_w // chunk
    mesh = plsc.VectorSubcoreMesh(core_axis_name="c", subcore_axis_name="s")

    def body(x_hbm, codes_hbm, out_hbm, codes_v, in_v, out_v):
        wid = lax.axis_index("s") * nc + lax.axis_index("c")
        pltpu.sync_copy(codes_hbm, codes_v)

        def do_chunk(j, carry):
            base = wid * per_w + j * chunk
            pltpu.sync_copy(x_hbm.at[pl.ds(base, chunk)], in_v)

            def do_vec(i, c):
                xv = in_v[pl.ds(i * LANES, LANES)]
                xc = jnp.minimum(jnp.maximum(xv, 0.0), float(NUM_CODES - 1))
                il = xc.astype(jnp.int32)
                ir = jnp.minimum(il + 1, NUM_CODES - 1)
                gl = plsc.load_gather(codes_v, [il])
                gr = plsc.load_gather(codes_v, [ir])
                ilf = il.astype(jnp.float32)
                out_v[pl.ds(i * LANES, LANES)] = (
                    gl * (1.0 - xv + ilf) + gr * (xv - ilf)
                )
                return c

            lax.fori_loop(0, chunk // LANES, do_vec, 0)
            pltpu.sync_copy(out_v, out_hbm.at[pl.ds(base, chunk)])
            return carry

        lax.fori_loop(0, n_chunks, do_chunk, 0)

    return pl.kernel(
        body,
        out_type=jax.ShapeDtypeStruct((n_total,), jnp.float32),
        mesh=mesh,
        scratch_types=[
            pltpu.VMEM((NUM_CODES,), jnp.float32),
            pltpu.VMEM((chunk,), jnp.float32),
            pltpu.VMEM((chunk,), jnp.float32),
        ],
        compiler_params=pltpu.CompilerParams(needs_layout_passes=False),
    )


def kernel(x, codes):
    shape = x.shape
    n_total = x.size
    x_flat = x.reshape(n_total)
    codes_flat = codes.reshape(NUM_CODES)
    chunk = 16384
    while n_total % (32 * chunk) != 0:
        chunk //= 2
    out = _build_sc_kernel(n_total, chunk)(x_flat, codes_flat)
    return out.reshape(shape)


# speedup vs baseline: 7873.4560x; 1.9915x over previous
import functools

import jax
import jax.numpy as jnp
from jax import lax
from jax.experimental import pallas as pl
from jax.experimental.pallas import tpu as pltpu
from jax.experimental.pallas import tpu_sc as plsc

NUM_CODES = 128
LANES = 16
NBUF = 2
UNROLL = 8


@functools.lru_cache(maxsize=None)
def _build_sc_kernel(n_total: int, chunk: int):
    info = plsc.get_sparse_core_info()
    nc, ns = info.num_cores, info.num_subcores
    nw = nc * ns
    per_w = n_total // nw
    assert n_total % nw == 0 and per_w % chunk == 0
    n_chunks = per_w // chunk
    assert n_chunks % NBUF == 0
    mesh = plsc.VectorSubcoreMesh(core_axis_name="c", subcore_axis_name="s")

    def body(x_hbm, codes_hbm, out_hbm, codes_v, in_v, out_v, *sems):
        in_sems, out_sems = sems[:NBUF], sems[NBUF:]
        wid = lax.axis_index("s") * nc + lax.axis_index("c")
        base_w = wid * per_w
        pltpu.sync_copy(codes_hbm, codes_v)

        def start_in(j, b):
            pltpu.async_copy(
                x_hbm.at[pl.ds(base_w + j * chunk, chunk)],
                in_v.at[pl.ds(b * chunk, chunk)], in_sems[b])

        def wait_in(b):
            pltpu.make_async_copy(
                x_hbm.at[pl.ds(base_w, chunk)],
                in_v.at[pl.ds(b * chunk, chunk)], in_sems[b]).wait()

        def start_out(j, b):
            pltpu.async_copy(
                out_v.at[pl.ds(b * chunk, chunk)],
                out_hbm.at[pl.ds(base_w + j * chunk, chunk)], out_sems[b])

        def wait_out(b):
            pltpu.make_async_copy(
                out_v.at[pl.ds(b * chunk, chunk)],
                out_hbm.at[pl.ds(base_w, chunk)], out_sems[b]).wait()

        for b in range(NBUF):
            start_in(b, b)

        @pl.loop(0, n_chunks, step=NBUF)
        def _(g):
            for b in range(NBUF):
                j = g + b
                wait_in(b)

                @pl.when(j >= NBUF)
                def _():
                    wait_out(b)

                boff = b * chunk

                @plsc.parallel_loop(0, chunk // LANES, unroll=UNROLL)
                def _(i):
                    xv = in_v[pl.ds(boff + i * LANES, LANES)]
                    xc = jnp.minimum(jnp.maximum(xv, 0.0),
                                     float(NUM_CODES - 1))
                    il = xc.astype(jnp.int32)
                    ir = jnp.minimum(il + 1, NUM_CODES - 1)
                    gl = plsc.load_gather(codes_v, [il])
                    gr = plsc.load_gather(codes_v, [ir])
                    ilf = il.astype(jnp.float32)
                    out_v[pl.ds(boff + i * LANES, LANES)] = (
                        gl * (1.0 - xv + ilf) + gr * (xv - ilf))

                start_out(j, b)

                @pl.when(j + NBUF < n_chunks)
                def _():
                    start_in(j + NBUF, b)

        for b in range(NBUF):
            wait_out(b)

    return pl.kernel(
        body,
        out_type=jax.ShapeDtypeStruct((n_total,), jnp.float32),
        mesh=mesh,
        scratch_types=[
            pltpu.VMEM((NUM_CODES,), jnp.float32),
            pltpu.VMEM((NBUF * chunk,), jnp.float32),
            pltpu.VMEM((NBUF * chunk,), jnp.float32),
        ] + [pltpu.SemaphoreType.DMA] * (2 * NBUF),
        compiler_params=pltpu.CompilerParams(needs_layout_passes=False),
    )


def kernel(x, codes):
    shape = x.shape
    n_total = x.size
    x_flat = x.reshape(n_total)
    codes_flat = codes.reshape(NUM_CODES)
    chunk = 16384
    while n_total % (32 * chunk * NBUF) != 0:
        chunk //= 2
    out = _build_sc_kernel(n_total, chunk)(x_flat, codes_flat)
    return out.reshape(shape)
